# baseline (speedup 1.0000x reference)
import functools

import jax
import jax.numpy as jnp
from jax import lax
from jax.experimental import pallas as pl
from jax.experimental.pallas import tpu as pltpu
from jax.experimental.pallas import tpu_sc as plsc

N_NODES = 100000
N_EDGES = 6400000
IN_CH = 10
HID = 16
PAD_CH = 16

NC = 2
NS = 16
NW = NC * NS
K = 400
CHUNKS = N_EDGES // K
CH_PW = CHUNKS // NW
STRIPE = 6256
N_PAD = STRIPE * NS


def _sc_accumulate(xpad, src2, dst2, zeros_stripe):
    mesh = plsc.VectorSubcoreMesh(core_axis_name="c", subcore_axis_name="s")

    @functools.partial(
        pl.kernel,
        out_type=jax.ShapeDtypeStruct((NC, N_PAD, PAD_CH), jnp.float32),
        mesh=mesh,
        scratch_types=[
            pltpu.VMEM((4, K), jnp.int32),
            pltpu.VMEM((4, K), jnp.int32),
            pltpu.VMEM((4, K, PAD_CH), jnp.float32),
            pltpu.VMEM_SHARED((N_PAD, PAD_CH), jnp.float32),
            pltpu.SemaphoreType.DMA((4,)),
            pltpu.SemaphoreType.DMA((4,)),
            pltpu.SemaphoreType.DMA((4,)),
        ],
        compiler_params=pltpu.CompilerParams(use_tc_tiling_on_sc=False),
    )
    def sck(xpad_hbm, src_hbm, dst_hbm, zeros_hbm, out_hbm,
            sv, dv, rows, accum, isem, gsem, ssem):
        c = lax.axis_index("c")
        s = lax.axis_index("s")
        w = s * NC + c

        pltpu.sync_copy(zeros_hbm, accum.at[pl.ds(s * STRIPE, STRIPE)])
        plsc.subcore_barrier()

        start = w * CH_PW

        def fire_idx(i, lc):
            pltpu.async_copy(src_hbm.at[start + lc], sv.at[i], isem.at[i])
            pltpu.async_copy(dst_hbm.at[start + lc], dv.at[i], isem.at[i])

        def drain_idx(i):
            pltpu.make_async_copy(src_hbm.at[0], sv.at[i], isem.at[i]).wait()
            pltpu.make_async_copy(src_hbm.at[0], dv.at[i], isem.at[i]).wait()

        def fire_gath(i):
            pltpu.async_copy(xpad_hbm.at[sv.at[i]], rows.at[i], gsem.at[i])

        def fire_scat(i):
            pltpu.async_copy(rows.at[i], accum.at[dv.at[i]], ssem.at[i],
                             add=True)

        def drain_rows(i, sem):
            pltpu.make_async_copy(xpad_hbm.at[pl.ds(0, K)], rows.at[i],
                                  sem.at[i]).wait()

        def step(lc, q, scat_drain, idx_fire, gath_fire):
            i, j1, j2 = q, (q + 1) % 4, (q + 2) % 4
            drain_rows(i, gsem)
            fire_scat(i)
            if scat_drain:
                drain_rows(j2, ssem)
            if idx_fire:
                fire_idx(j2, lc + 2)
            if gath_fire:
                drain_idx(j1)
                fire_gath(j1)

        fire_idx(0, 0)
        fire_idx(1, 1)
        drain_idx(0)
        fire_gath(0)
        step(0, 0, False, True, True)
        step(1, 1, False, True, True)
        step(2, 2, True, True, True)
        step(3, 3, True, True, True)

        @pl.loop(1, CH_PW // 4 - 1)
        def _(t):
            lc = 4 * t
            step(lc + 0, 0, True, True, True)
            step(lc + 1, 1, True, True, True)
            step(lc + 2, 2, True, True, True)
            step(lc + 3, 3, True, True, True)

        last = CH_PW - 4
        step(last + 0, 0, True, True, True)
        step(last + 1, 1, True, True, True)
        step(last + 2, 2, True, False, True)
        step(last + 3, 3, True, False, False)
        drain_rows(2, ssem)
        drain_rows(3, ssem)

        plsc.subcore_barrier()
        pltpu.sync_copy(accum.at[pl.ds(s * STRIPE, STRIPE)],
                        out_hbm.at[c, pl.ds(s * STRIPE, STRIPE)])

    return sck(xpad, src2, dst2, zeros_stripe)


def _tc_finish_body(p_ref, x_ref, wl_ref, wr_ref, bl_ref, o_ref):
    sums = p_ref[0] + p_ref[1]
    cnt = jnp.maximum(sums[:, IN_CH:IN_CH + 1], 1.0)
    mean = sums[:, :IN_CH] / cnt
    o_ref[...] = (
        jnp.dot(mean, wl_ref[...], preferred_element_type=jnp.float32)
        + bl_ref[...]
        + jnp.dot(x_ref[...], wr_ref[...], preferred_element_type=jnp.float32)
    )


def _tc_finish(partial, x, W_l, W_r, b_l):
    B = 4000
    grid = (N_NODES // B,)
    return pl.pallas_call(
        _tc_finish_body,
        grid=grid,
        in_specs=[
            pl.BlockSpec((NC, B, PAD_CH), lambda i: (0, i, 0)),
            pl.BlockSpec((B, IN_CH), lambda i: (i, 0)),
            pl.BlockSpec((IN_CH, HID), lambda i: (0, 0)),
            pl.BlockSpec((IN_CH, HID), lambda i: (0, 0)),
            pl.BlockSpec((1, HID), lambda i: (0, 0)),
        ],
        out_specs=pl.BlockSpec((B, HID), lambda i: (i, 0)),
        out_shape=jax.ShapeDtypeStruct((N_NODES, HID), jnp.float32),
    )(partial, x, W_l, W_r, b_l.reshape(1, HID))


def kernel(x, edge_index, W_l, W_r, b_l):
    src = edge_index[0].astype(jnp.int32).reshape(CHUNKS, K)
    dst = edge_index[1].astype(jnp.int32).reshape(CHUNKS, K)
    xpad = jnp.concatenate(
        [x,
         jnp.ones((N_NODES, 1), jnp.float32),
         jnp.zeros((N_NODES, PAD_CH - IN_CH - 1), jnp.float32)],
        axis=1,
    )
    zeros_stripe = jnp.zeros((STRIPE, PAD_CH), jnp.float32)
    partial = _sc_accumulate(xpad, src, dst, zeros_stripe)
    return _tc_finish(partial, x, W_l, W_r, b_l)
